# baseline (speedup 1.0000x reference)
import jax
import jax.numpy as jnp
from jax import lax
from jax.experimental import pallas as pl
from jax.experimental.pallas import tpu as pltpu
from jax.experimental.pallas import tpu_sc as plsc

BATCH = 4096
HIST = 50
D_MODEL = 512
TOTAL = BATCH * HIST

NUM_CORES = 2
NUM_SUBCORES = 16
NUM_WORKERS = NUM_CORES * NUM_SUBCORES
ROWS_PER_W = TOTAL // NUM_WORKERS

CHUNK = 40
NCHUNK = ROWS_PER_W // CHUNK
NSLOT = 6
LOOKAHEAD = 3


def _emb_body(idx_hbm, table_hbm, out_hbm, idx_v, bufs, gsems, ssems):
    wid = lax.axis_index("s") * NUM_CORES + lax.axis_index("c")
    base = wid * ROWS_PER_W

    pltpu.sync_copy(idx_hbm.at[pl.ds(base, ROWS_PER_W)], idx_v)

    def _gather(k, b):
        return pltpu.make_async_copy(
            table_hbm.at[idx_v.at[pl.ds(k * CHUNK, CHUNK)]], bufs[b], gsems[b])

    def _store(k, b):
        return pltpu.make_async_copy(
            bufs[b], out_hbm.at[pl.ds(base + k * CHUNK, CHUNK)], ssems[b])

    for k in range(LOOKAHEAD):
        _gather(k, k).start()

    def _iter(k, carry):
        slot = lax.rem(k, NSLOT)

        def _run(b):
            _gather(k, b).wait()
            _store(k, b).start()

            bn = (b + LOOKAHEAD) % NSLOT

            @pl.when(k >= NSLOT - LOOKAHEAD)
            def _():
                _store(k - (NSLOT - LOOKAHEAD), bn).wait()

            @pl.when(k + LOOKAHEAD < NCHUNK)
            def _():
                _gather(k + LOOKAHEAD, bn).start()

        for b in range(NSLOT):
            @pl.when(slot == b)
            def _(b=b):
                _run(b)

        return carry

    lax.fori_loop(0, NCHUNK, _iter, 0)

    for k in range(NCHUNK - (NSLOT - LOOKAHEAD), NCHUNK):
        _store(k, k % NSLOT).wait()


@jax.jit
def _embed(idx_flat, table):
    mesh = plsc.VectorSubcoreMesh(core_axis_name="c", subcore_axis_name="s")
    run = pl.kernel(
        _emb_body,
        mesh=mesh,
        out_type=jax.ShapeDtypeStruct((TOTAL, D_MODEL), jnp.float32),
        scratch_types=[
            pltpu.VMEM((ROWS_PER_W,), jnp.int32),
            tuple(pltpu.VMEM((CHUNK, D_MODEL), jnp.float32)
                  for _ in range(NSLOT)),
            tuple(pltpu.SemaphoreType.DMA for _ in range(NSLOT)),
            tuple(pltpu.SemaphoreType.DMA for _ in range(NSLOT)),
        ],
    )
    return run(idx_flat, table)


def kernel(x, table):
    idx_t = x.astype(jnp.int32).T.reshape(-1)
    out = _embed(idx_t, table)
    return out.reshape(HIST, BATCH, D_MODEL).transpose(1, 0, 2)


# speedup vs baseline: 1.0039x; 1.0039x over previous
import jax
import jax.numpy as jnp
from jax import lax
from jax.experimental import pallas as pl
from jax.experimental.pallas import tpu as pltpu
from jax.experimental.pallas import tpu_sc as plsc

BATCH = 4096
HIST = 50
D_MODEL = 512
TOTAL = BATCH * HIST

NUM_CORES = 2
NUM_SUBCORES = 16
NUM_WORKERS = NUM_CORES * NUM_SUBCORES
ROWS_PER_W = TOTAL // NUM_WORKERS

CHUNK = 80
NCHUNK = ROWS_PER_W // CHUNK
NSLOT = 3
LOOKAHEAD = 2


def _emb_body(idx_hbm, table_hbm, out_hbm, idx_v, bufs, gsems, ssems):
    wid = lax.axis_index("s") * NUM_CORES + lax.axis_index("c")
    base = wid * ROWS_PER_W

    pltpu.sync_copy(idx_hbm.at[pl.ds(base, ROWS_PER_W)], idx_v)

    def _gather(k, b):
        return pltpu.make_async_copy(
            table_hbm.at[idx_v.at[pl.ds(k * CHUNK, CHUNK)]], bufs[b], gsems[b])

    def _store(k, b):
        return pltpu.make_async_copy(
            bufs[b], out_hbm.at[pl.ds(base + k * CHUNK, CHUNK)], ssems[b])

    for k in range(LOOKAHEAD):
        _gather(k, k).start()

    def _iter(k, carry):
        slot = lax.rem(k, NSLOT)

        def _run(b):
            _gather(k, b).wait()
            _store(k, b).start()

            bn = (b + LOOKAHEAD) % NSLOT

            @pl.when(k >= NSLOT - LOOKAHEAD)
            def _():
                _store(k - (NSLOT - LOOKAHEAD), bn).wait()

            @pl.when(k + LOOKAHEAD < NCHUNK)
            def _():
                _gather(k + LOOKAHEAD, bn).start()

        for b in range(NSLOT):
            @pl.when(slot == b)
            def _(b=b):
                _run(b)

        return carry

    lax.fori_loop(0, NCHUNK, _iter, 0)

    for k in range(NCHUNK - (NSLOT - LOOKAHEAD), NCHUNK):
        _store(k, k % NSLOT).wait()


@jax.jit
def _embed(idx_flat, table):
    mesh = plsc.VectorSubcoreMesh(core_axis_name="c", subcore_axis_name="s")
    run = pl.kernel(
        _emb_body,
        mesh=mesh,
        out_type=jax.ShapeDtypeStruct((TOTAL, D_MODEL), jnp.float32),
        scratch_types=[
            pltpu.VMEM((ROWS_PER_W,), jnp.int32),
            tuple(pltpu.VMEM((CHUNK, D_MODEL), jnp.float32)
                  for _ in range(NSLOT)),
            tuple(pltpu.SemaphoreType.DMA for _ in range(NSLOT)),
            tuple(pltpu.SemaphoreType.DMA for _ in range(NSLOT)),
        ],
    )
    return run(idx_flat, table)


def kernel(x, table):
    idx_t = x.astype(jnp.int32).T.reshape(-1)
    out = _embed(idx_t, table)
    return out.reshape(HIST, BATCH, D_MODEL).transpose(1, 0, 2)
